# Initial kernel scaffold; baseline (speedup 1.0000x reference)
import jax
import jax.numpy as jnp
from jax.experimental import pallas as pl


def kernel(x, edge_index, edge_attr, batch, c1_Wrel, c1_brel, c1_Wroot, bn1_g, bn1_b, p1_Wrel, p1_brel, p1_Wroot, c2_Wrel, c2_brel, c2_Wroot, bn2_g, bn2_b, p2_Wrel, p2_brel, p2_Wroot, c3_Wrel, c3_brel, c3_Wroot, bn3_g, bn3_b, p3_Wrel, p3_brel, p3_Wroot, lin1_W, lin1_b, lin2_W, lin2_b, lin3_W, lin3_b):
    raise NotImplementedError("write your pallas kernel here")


# speedup vs baseline: 1.4165x; 1.4165x over previous
import functools
import math

import jax
import jax.numpy as jnp
from jax.experimental import pallas as pl

N = 10000
E = 320000
H = 128


def _head_kernel(s_ref, w1_ref, b1_ref, w2_ref, b2_ref, w3_ref, b3_ref, o_ref):
    s = s_ref[...]
    h1 = jnp.maximum(s @ w1_ref[...].T + b1_ref[...], 0.0)
    h2 = jnp.maximum(h1 @ w2_ref[...].T + b2_ref[...], 0.0)
    o_ref[...] = h2 @ w3_ref[...].T + b3_ref[...]


def kernel(x, edge_index, edge_attr, batch,
           c1_Wrel, c1_brel, c1_Wroot, bn1_g, bn1_b, p1_Wrel, p1_brel, p1_Wroot,
           c2_Wrel, c2_brel, c2_Wroot, bn2_g, bn2_b, p2_Wrel, p2_brel, p2_Wroot,
           c3_Wrel, c3_brel, c3_Wroot, bn3_g, bn3_b, p3_Wrel, p3_brel, p3_Wroot,
           lin1_W, lin1_b, lin2_W, lin2_b, lin3_W, lin3_b):
    src = edge_index[0]
    dst = edge_index[1]
    k1 = math.ceil(0.5 * N)
    k2 = math.ceil(0.5 * k1)
    k3 = math.ceil(0.5 * k2)

    esrc, edst = src, dst
    m = jnp.ones((N,), jnp.float32)
    h = x
    n_live = N
    readouts = []
    params = [
        (c1_Wrel, c1_brel, c1_Wroot, bn1_g, bn1_b, p1_Wrel, p1_brel, p1_Wroot, k1),
        (c2_Wrel, c2_brel, c2_Wroot, bn2_g, bn2_b, p2_Wrel, p2_brel, p2_Wroot, k2),
        (c3_Wrel, c3_brel, c3_Wroot, bn3_g, bn3_b, p3_Wrel, p3_brel, p3_Wroot, k3),
    ]
    for li, (Wrel, brel, Wroot, g, b, pWrel, pbrel, pWroot, k) in enumerate(params):
        agg = jax.ops.segment_sum(h[esrc], edst, num_segments=N + 1)[:N]
        t = agg @ Wrel.T + brel + h @ Wroot.T
        mu = jnp.sum(t * m[:, None], axis=0) / n_live
        var = jnp.sum(t * t * m[:, None], axis=0) / n_live - mu * mu
        h = jnp.maximum(g * (t - mu) * jax.lax.rsqrt(var + 1e-5) + b, 0.0) * m[:, None]
        sagg2 = jax.ops.segment_sum(h[esrc], edst, num_segments=N + 1)[:N]
        score = (sagg2 @ pWrel.T)[:, 0] + pbrel[0] + (h @ pWroot.T)[:, 0]
        score = jnp.where(m > 0, score, -jnp.inf)
        topv = jax.lax.top_k(score, k)[0]
        thr = topv[k - 1]
        m_new = (score >= thr).astype(jnp.float32)
        h = h * (jnp.tanh(score) * m_new)[:, None]
        mean_r = jnp.sum(h, axis=0, keepdims=True) / k
        max_r = jnp.max(jnp.where(m_new[:, None] > 0, h, -jnp.inf), axis=0, keepdims=True)
        readouts.append(jnp.concatenate([mean_r, max_r], axis=1))
        if li < 2:
            m_ext = jnp.concatenate([m_new, jnp.zeros((1,), jnp.float32)])
            live = m_new[esrc] * m_ext[edst]
            esrc = jnp.where(live > 0, esrc, 0)
            edst = jnp.where(live > 0, edst, N)
            m = m_new
            n_live = k

    s = readouts[0] + readouts[1] + readouts[2]
    out = pl.pallas_call(
        _head_kernel,
        out_shape=jax.ShapeDtypeStruct((1, 2), jnp.float32),
    )(s, lin1_W, lin1_b, lin2_W, lin2_b, lin3_W, lin3_b)
    return out


# speedup vs baseline: 7.8854x; 5.5670x over previous
import functools
import math

import jax
import jax.numpy as jnp
from jax import lax
from jax.experimental import pallas as pl
from jax.experimental.pallas import tpu as pltpu
from jax.experimental.pallas import tpu_sc as plsc

N = 10000
E = 320000
H = 128
NC = 2
NS = 16
NW = NC * NS
NPAD = 10112
ROWS_PER_TILE = NPAD // NS
EPT = 10240
E_PAD = EPT * NW
CH = 128
NCHUNK = EPT // CH
F32 = jnp.float32
BIG_NEG = -3.0e38


def _segsum_body(src_hbm, dst_hbm, h_hbm, out_hbm,
                 src_v0, dst_v0, rows_v0, src_v1, dst_v1, rows_v1,
                 zbuf, acc, sem0, sem1):
    cid = lax.axis_index("c")
    sid = lax.axis_index("s")
    wid = sid * NC + cid
    ebase = wid * EPT

    def _zrow(r, _):
        for j in range(H // 16):
            zbuf[r, pl.ds(16 * j, 16)] = jnp.zeros((16,), F32)
        return 0
    lax.fori_loop(0, CH, _zrow, 0)
    r0 = sid * ROWS_PER_TILE
    for off, nn in ((0, 128), (128, 128), (256, 128), (384, 128), (512, 120)):
        pltpu.sync_copy(zbuf.at[pl.ds(0, nn)], acc.at[pl.ds(r0 + off, nn)])
    plsc.subcore_barrier()

    bufs = ((src_v0, dst_v0, rows_v0, sem0), (src_v1, dst_v1, rows_v1, sem1))

    def _fire(i, b):
        src_v, dst_v, rows_v, sem = bufs[b]
        pltpu.sync_copy(src_hbm.at[pl.ds(ebase + i * CH, CH)], src_v)
        pltpu.sync_copy(dst_hbm.at[pl.ds(ebase + i * CH, CH)], dst_v)
        pltpu.make_async_copy(h_hbm.at[src_v], rows_v, sem).start()

    def _drain(b):
        src_v, dst_v, rows_v, sem = bufs[b]
        pltpu.make_async_copy(h_hbm.at[src_v], rows_v, sem).wait()
        pltpu.sync_copy(rows_v, acc.at[dst_v], add=True)

    _fire(0, 0)

    def _step(it, _):
        _fire(2 * it + 1, 1)
        _drain(0)

        @pl.when(it < NCHUNK // 2 - 1)
        def _():
            _fire(2 * it + 2, 0)
        _drain(1)
        return 0

    lax.fori_loop(0, NCHUNK // 2, _step, 0)
    plsc.subcore_barrier()

    for off, nn in ((0, 128), (128, 128), (256, 128), (384, 128), (512, 120)):
        pltpu.sync_copy(acc.at[pl.ds(r0 + off, nn)],
                        out_hbm.at[cid, pl.ds(r0 + off, nn)])


@functools.lru_cache(maxsize=1)
def _segsum_sc_kernel():
    mesh = plsc.VectorSubcoreMesh(core_axis_name="c", subcore_axis_name="s")
    return pl.kernel(
        _segsum_body, mesh=mesh,
        out_type=jax.ShapeDtypeStruct((NC, NPAD, H), F32),
        scratch_types=[
            pltpu.VMEM((CH,), jnp.int32), pltpu.VMEM((CH,), jnp.int32),
            pltpu.VMEM((CH, H), F32),
            pltpu.VMEM((CH,), jnp.int32), pltpu.VMEM((CH,), jnp.int32),
            pltpu.VMEM((CH, H), F32),
            pltpu.VMEM((CH, H), F32),
            pltpu.VMEM_SHARED((NPAD, H), F32),
            pltpu.SemaphoreType.DMA, pltpu.SemaphoreType.DMA,
        ],
    )


def _segsum_sc(esrc, edst, h):
    return _segsum_sc_kernel()(esrc, edst, h)


def _conv_bn_body(n_live, p_ref, h_ref, wrel_ref, brel_ref, wroot_ref,
                  g_ref, b_ref, m_ref, out_ref):
    agg = p_ref[0] + p_ref[1]
    t = (jnp.dot(agg, wrel_ref[...].T, preferred_element_type=F32)
         + brel_ref[...]
         + jnp.dot(h_ref[...], wroot_ref[...].T, preferred_element_type=F32))
    m = m_ref[...]
    mu = jnp.sum(t * m, axis=0, keepdims=True) / n_live
    d = t - mu
    var = jnp.sum(d * d * m, axis=0, keepdims=True) / n_live
    hn = g_ref[...] * d * lax.rsqrt(var + 1e-5) + b_ref[...]
    out_ref[...] = jnp.maximum(hn, 0.0) * m


def _conv_bn_tc(n_live):
    return pl.pallas_call(
        functools.partial(_conv_bn_body, n_live),
        out_shape=jax.ShapeDtypeStruct((NPAD, H), F32),
    )


def _score_pool_body(k, p_ref, h_ref, pwrel_ref, pbrel_ref, pwroot_ref,
                     m_ref, hout_ref, mout_ref, ro_ref):
    sg = p_ref[0] + p_ref[1]
    h = h_ref[...]
    col0 = lax.broadcasted_iota(jnp.int32, (H, H), 1) == 0
    wm = jnp.where(col0, pwrel_ref[...].T, 0.0) + jnp.zeros((H, H), F32)
    wr = jnp.where(col0, pwroot_ref[...].T, 0.0) + jnp.zeros((H, H), F32)
    score = (jnp.dot(sg, wm, preferred_element_type=F32)[:, :1]
             + pbrel_ref[0]
             + jnp.dot(h, wr, preferred_element_type=F32)[:, :1])
    m = m_ref[...]
    score = jnp.where(m > 0, score, BIG_NEG)

    b = lax.bitcast_convert_type(score, jnp.uint32)
    flip = jnp.where(b >> 31 == 1, jnp.uint32(0xFFFFFFFF), jnp.uint32(0x80000000))
    ukey = b ^ flip

    def _vbit(i, pfx):
        cand = pfx | (jnp.uint32(1) << (jnp.uint32(31) - i.astype(jnp.uint32)))
        cnt = jnp.sum((ukey >= cand).astype(jnp.int32))
        return jnp.where(cnt >= k, cand, pfx)
    thr = lax.fori_loop(0, 32, _vbit, jnp.uint32(0))

    above = ukey > thr
    tie = ukey == thr
    g = jnp.sum(above.astype(jnp.int32))
    r = k - g
    idx = lax.broadcasted_iota(jnp.int32, (NPAD, 1), 0)

    def _ibit(i, pfx):
        cand = pfx | (jnp.int32(1) << (jnp.int32(13) - i))
        cnt = jnp.sum((tie & (idx < cand)).astype(jnp.int32))
        return jnp.where(cnt < r, cand, pfx)
    cut = lax.fori_loop(0, 14, _ibit, jnp.int32(0))

    m_new = (above | (tie & (idx <= cut))).astype(F32)
    gate = jnp.tanh(score) * m_new
    h_out = h * gate
    hout_ref[...] = h_out
    mout_ref[...] = m_new
    mean_r = jnp.sum(h_out, axis=0, keepdims=True) / k
    max_r = jnp.max(jnp.where(m_new > 0, h_out, BIG_NEG), axis=0, keepdims=True)
    ro_ref[...] = jnp.concatenate([mean_r, max_r], axis=1)


def _score_pool_tc(k):
    return pl.pallas_call(
        functools.partial(_score_pool_body, k),
        out_shape=(
            jax.ShapeDtypeStruct((NPAD, H), F32),
            jax.ShapeDtypeStruct((NPAD, 1), F32),
            jax.ShapeDtypeStruct((1, 2 * H), F32),
        ),
    )


def _head_body(r1_ref, r2_ref, r3_ref, w1_ref, b1_ref, w2_ref, b2_ref,
               w3_ref, b3_ref, o_ref):
    s = r1_ref[...] + r2_ref[...] + r3_ref[...]
    h1 = jnp.maximum(jnp.dot(s, w1_ref[...].T, preferred_element_type=F32)
                     + b1_ref[...], 0.0)
    h2 = jnp.maximum(jnp.dot(h1, w2_ref[...].T, preferred_element_type=F32)
                     + b2_ref[...], 0.0)
    o_ref[...] = jnp.dot(h2, w3_ref[...].T, preferred_element_type=F32) + b3_ref[...]


def kernel(x, edge_index, edge_attr, batch,
           c1_Wrel, c1_brel, c1_Wroot, bn1_g, bn1_b, p1_Wrel, p1_brel, p1_Wroot,
           c2_Wrel, c2_brel, c2_Wroot, bn2_g, bn2_b, p2_Wrel, p2_brel, p2_Wroot,
           c3_Wrel, c3_brel, c3_Wroot, bn3_g, bn3_b, p3_Wrel, p3_brel, p3_Wroot,
           lin1_W, lin1_b, lin2_W, lin2_b, lin3_W, lin3_b):
    k1 = math.ceil(0.5 * N)
    k2 = math.ceil(0.5 * k1)
    k3 = math.ceil(0.5 * k2)

    esrc = jnp.concatenate([edge_index[0],
                            jnp.full((E_PAD - E,), N, jnp.int32)])
    edst = jnp.concatenate([edge_index[1],
                            jnp.full((E_PAD - E,), NPAD - 1, jnp.int32)])
    h = jnp.concatenate([x, jnp.zeros((NPAD - N, H), F32)], axis=0)
    m = jnp.concatenate([jnp.ones((N, 1), F32), jnp.zeros((NPAD - N, 1), F32)])

    params = [
        (c1_Wrel, c1_brel, c1_Wroot, bn1_g, bn1_b, p1_Wrel, p1_brel, p1_Wroot, N, k1),
        (c2_Wrel, c2_brel, c2_Wroot, bn2_g, bn2_b, p2_Wrel, p2_brel, p2_Wroot, k1, k2),
        (c3_Wrel, c3_brel, c3_Wroot, bn3_g, bn3_b, p3_Wrel, p3_brel, p3_Wroot, k2, k3),
    ]
    readouts = []
    for Wrel, brel, Wroot, g, b, pWrel, pbrel, pWroot, n_live, k in params:
        p = _segsum_sc(esrc, edst, h)
        h = _conv_bn_tc(n_live)(p, h, Wrel, brel.reshape(1, H), Wroot,
                                g.reshape(1, H), b.reshape(1, H), m)
        sp = _segsum_sc(esrc, edst, h)
        h, m, ro = _score_pool_tc(k)(sp, h, pWrel, pbrel, pWroot, m)
        readouts.append(ro)

    out = pl.pallas_call(
        _head_body,
        out_shape=jax.ShapeDtypeStruct((1, 2), F32),
    )(readouts[0], readouts[1], readouts[2],
      lin1_W, lin1_b.reshape(1, H), lin2_W, lin2_b.reshape(1, H // 2),
      lin3_W, lin3_b.reshape(1, 2))
    return out
